# baseline (speedup 1.0000x reference)
import functools

import jax
import jax.numpy as jnp
from jax.experimental import pallas as pl
from jax.experimental.pallas import tpu as pltpu


def _tc_body(n_cls, nb, nj, n_total, x_ref, t_ref, out_ref,
             cnt_ref, fn_ref, ces_ref):
    b = pl.program_id(0)
    j = pl.program_id(1)

    @pl.when(jnp.logical_and(b == 0, j == 0))
    def _init():
        cnt_ref[...] = jnp.zeros_like(cnt_ref)
        fn_ref[...] = jnp.zeros_like(fn_ref)
        ces_ref[...] = jnp.zeros_like(ces_ref)

    x = x_ref[0]
    t = t_ref[...]

    m = jnp.max(x, axis=0, keepdims=True)
    xm = x - m
    e = jnp.exp(xm)
    s = jnp.sum(e, axis=0, keepdims=True)
    lse = m + jnp.log(s)

    cls = jax.lax.broadcasted_iota(jnp.int32, x.shape, 0)
    ohf = (cls == t).astype(jnp.float32)
    xt = jnp.sum(x * ohf, axis=0, keepdims=True)
    ce = lse - xt

    am = jnp.min(jnp.where(xm == 0.0, cls, n_cls), axis=0, keepdims=True)
    wrongf = (am != t).astype(jnp.float32)

    cnt_ref[...] += jnp.sum(ohf, axis=(1, 2), keepdims=True)
    fn_ref[...] += jnp.sum(ohf * wrongf, axis=(1, 2), keepdims=True)
    ces_ref[...] += jnp.sum(ohf * ce, axis=(1, 2), keepdims=True)

    @pl.when(jnp.logical_and(b == nb - 1, j == nj - 1))
    def _fin():
        cnt = cnt_ref[...]
        fn = fn_ref[...]
        ces = ces_ref[...]
        gt_c = jnp.where(cnt > 0, cnt, 1.0)
        fn_c = jnp.where(fn > 0, fn, 1.0)
        loss = jnp.sum((fn_c / gt_c) * ces) / jnp.float32(n_total)
        out_ref[...] = jnp.full(out_ref.shape, loss, jnp.float32)


def kernel(input, target):
    nb, n_cls, h, w = input.shape
    rows = 128
    nj = h // rows
    n_total = nb * h * w

    body = functools.partial(_tc_body, n_cls, nb, nj, n_total)

    out = pl.pallas_call(
        body,
        grid=(nb, nj),
        in_specs=[
            pl.BlockSpec((1, n_cls, rows, w), lambda b, j: (b, 0, j, 0)),
            pl.BlockSpec((1, rows, w), lambda b, j: (b, j, 0)),
        ],
        out_specs=pl.BlockSpec((8, 128), lambda b, j: (0, 0)),
        out_shape=jax.ShapeDtypeStruct((8, 128), jnp.float32),
        scratch_shapes=[
            pltpu.VMEM((n_cls, 1, 1), jnp.float32),
            pltpu.VMEM((n_cls, 1, 1), jnp.float32),
            pltpu.VMEM((n_cls, 1, 1), jnp.float32),
        ],
        compiler_params=pltpu.CompilerParams(
            dimension_semantics=("arbitrary", "arbitrary"),
        ),
    )(input, target)
    return out[0, 0]


# speedup vs baseline: 1.0339x; 1.0339x over previous
import functools

import jax
import jax.numpy as jnp
from jax.experimental import pallas as pl
from jax.experimental.pallas import tpu as pltpu


def _tc_body(n_cls, nb, nj, n_total, x_ref, t_ref, out_ref,
             cnt_ref, fn_ref, ces_ref):
    b = pl.program_id(0)
    j = pl.program_id(1)

    @pl.when(jnp.logical_and(b == 0, j == 0))
    def _init():
        cnt_ref[...] = jnp.zeros_like(cnt_ref)
        fn_ref[...] = jnp.zeros_like(fn_ref)
        ces_ref[...] = jnp.zeros_like(ces_ref)

    x = x_ref[0]
    t = t_ref[...]

    m = jnp.max(x, axis=0, keepdims=True)
    xm = x - m
    e = jnp.exp(xm)
    s = jnp.sum(e, axis=0, keepdims=True)
    lse = m + jnp.log(s)

    cls = jax.lax.broadcasted_iota(jnp.int32, x.shape, 0)
    ohf = (cls == t).astype(jnp.float32)
    xt = jnp.sum(x * ohf, axis=0, keepdims=True)
    ce = lse - xt

    am = jnp.min(jnp.where(xm == 0.0, cls, n_cls), axis=0, keepdims=True)
    wrongf = (am != t).astype(jnp.float32)

    cnt_ref[...] += jnp.sum(ohf, axis=(1, 2), keepdims=True)
    fn_ref[...] += jnp.sum(ohf * wrongf, axis=(1, 2), keepdims=True)
    ces_ref[...] += jnp.sum(ohf * ce, axis=(1, 2), keepdims=True)

    @pl.when(jnp.logical_and(b == nb - 1, j == nj - 1))
    def _fin():
        cnt = cnt_ref[...]
        fn = fn_ref[...]
        ces = ces_ref[...]
        gt_c = jnp.where(cnt > 0, cnt, 1.0)
        fn_c = jnp.where(fn > 0, fn, 1.0)
        loss = jnp.sum((fn_c / gt_c) * ces) / jnp.float32(n_total)
        out_ref[...] = jnp.full(out_ref.shape, loss, jnp.float32)


def kernel(input, target):
    nb, n_cls, h, w = input.shape
    rows = 32
    nj = h // rows
    n_total = nb * h * w

    body = functools.partial(_tc_body, n_cls, nb, nj, n_total)

    out = pl.pallas_call(
        body,
        grid=(nb, nj),
        in_specs=[
            pl.BlockSpec((1, n_cls, rows, w), lambda b, j: (b, 0, j, 0)),
            pl.BlockSpec((1, rows, w), lambda b, j: (b, j, 0)),
        ],
        out_specs=pl.BlockSpec((8, 128), lambda b, j: (0, 0)),
        out_shape=jax.ShapeDtypeStruct((8, 128), jnp.float32),
        scratch_shapes=[
            pltpu.VMEM((n_cls, 1, 1), jnp.float32),
            pltpu.VMEM((n_cls, 1, 1), jnp.float32),
            pltpu.VMEM((n_cls, 1, 1), jnp.float32),
        ],
        compiler_params=pltpu.CompilerParams(
            dimension_semantics=("arbitrary", "arbitrary"),
        ),
    )(input, target)
    return out[0, 0]


# speedup vs baseline: 1.3458x; 1.3017x over previous
import functools

import jax
import jax.numpy as jnp
from jax.experimental import pallas as pl
from jax.experimental.pallas import tpu as pltpu


def _tc_body(n_cls, nb, nj, n_total, x_ref, t_ref, out_ref,
             cnt_ref, fn_ref, ces_ref):
    b = pl.program_id(0)
    j = pl.program_id(1)

    @pl.when(jnp.logical_and(b == 0, j == 0))
    def _init():
        cnt_ref[...] = jnp.zeros_like(cnt_ref)
        fn_ref[...] = jnp.zeros_like(fn_ref)
        ces_ref[...] = jnp.zeros_like(ces_ref)

    x = x_ref[0]
    t = t_ref[...]

    m = jnp.max(x, axis=0, keepdims=True)
    xm = x - m
    e = jnp.exp(xm)
    s = jnp.sum(e, axis=0, keepdims=True)
    lse = m + jnp.log(s)

    cls = jax.lax.broadcasted_iota(jnp.int32, x.shape, 0)
    ohf = (cls == t).astype(jnp.float32)
    xt = jnp.sum(x * ohf, axis=0, keepdims=True)
    ce = lse - xt

    wrongf = (xt < m).astype(jnp.float32)

    cnt_ref[...] += jnp.sum(ohf, axis=(1, 2), keepdims=True)
    fn_ref[...] += jnp.sum(ohf * wrongf, axis=(1, 2), keepdims=True)
    ces_ref[...] += jnp.sum(ohf * ce, axis=(1, 2), keepdims=True)

    @pl.when(jnp.logical_and(b == nb - 1, j == nj - 1))
    def _fin():
        cnt = cnt_ref[...]
        fn = fn_ref[...]
        ces = ces_ref[...]
        gt_c = jnp.where(cnt > 0, cnt, 1.0)
        fn_c = jnp.where(fn > 0, fn, 1.0)
        loss = jnp.sum((fn_c / gt_c) * ces) / jnp.float32(n_total)
        out_ref[...] = jnp.full(out_ref.shape, loss, jnp.float32)


def kernel(input, target):
    nb, n_cls, h, w = input.shape
    rows = 64
    nj = h // rows
    n_total = nb * h * w

    body = functools.partial(_tc_body, n_cls, nb, nj, n_total)

    out = pl.pallas_call(
        body,
        grid=(nb, nj),
        in_specs=[
            pl.BlockSpec((1, n_cls, rows, w), lambda b, j: (b, 0, j, 0)),
            pl.BlockSpec((1, rows, w), lambda b, j: (b, j, 0)),
        ],
        out_specs=pl.BlockSpec((8, 128), lambda b, j: (0, 0)),
        out_shape=jax.ShapeDtypeStruct((8, 128), jnp.float32),
        scratch_shapes=[
            pltpu.VMEM((n_cls, 1, 1), jnp.float32),
            pltpu.VMEM((n_cls, 1, 1), jnp.float32),
            pltpu.VMEM((n_cls, 1, 1), jnp.float32),
        ],
        compiler_params=pltpu.CompilerParams(
            dimension_semantics=("arbitrary", "arbitrary"),
        ),
    )(input, target)
    return out[0, 0]
